# Initial kernel scaffold; baseline (speedup 1.0000x reference)
import jax
import jax.numpy as jnp
from jax.experimental import pallas as pl


def kernel(cls_tok, regs, patches2d, roi_side):
    raise NotImplementedError("write your pallas kernel here")


# speedup vs baseline: 1.3241x; 1.3241x over previous
import jax
import jax.numpy as jnp
from jax.experimental import pallas as pl
from jax.experimental.pallas import tpu as pltpu

_B, _R, _D = 64, 4, 768
_H = _W = 37
_HW = _H * _W


def _encoder_body(r_ref, cues_ref, patches_ref, out_ref):
    r = r_ref[0]
    cues = cues_ref[0]
    patches = patches_ref[0]
    c = cues.shape[0]

    sim = jax.lax.dot_general(
        cues, patches, (((1,), (1,)), ((), ())),
        preferred_element_type=jnp.float32)

    idx = jnp.argmax(sim, axis=1, keepdims=True)
    hh = idx // _W
    ww = idx % _W

    pos = jax.lax.broadcasted_iota(jnp.int32, (c, _HW), 1)
    rowp = pos // _W
    colp = pos % _W
    inside = (jnp.abs(rowp - hh) <= r) & (jnp.abs(colp - ww) <= r)
    maskf = jnp.where(inside, 1.0, 0.0)

    num = jax.lax.dot_general(
        maskf, patches, (((1,), (0,)), ((), ())),
        preferred_element_type=jnp.float32)
    cnt = jnp.sum(maskf, axis=1, keepdims=True)
    rois = num / cnt

    toks = jnp.concatenate([cues, rois], axis=0)
    nrm = jnp.sqrt(jnp.sum(toks * toks, axis=1, keepdims=True))
    out_ref[0] = toks / jnp.maximum(nrm, 1e-12)


def kernel(cls_tok, regs, patches2d, roi_side):
    b, h, w, d = patches2d.shape
    c = 1 + regs.shape[1]
    cues = jnp.concatenate([cls_tok[:, None, :], regs], axis=1)
    patches = patches2d.reshape(b, h * w, d)
    r = jnp.asarray(roi_side // 2, jnp.int32).reshape(1)

    out = pl.pallas_call(
        _encoder_body,
        grid=(b,),
        in_specs=[
            pl.BlockSpec(memory_space=pltpu.SMEM),
            pl.BlockSpec((1, c, d), lambda i: (i, 0, 0)),
            pl.BlockSpec((1, h * w, d), lambda i: (i, 0, 0)),
        ],
        out_specs=pl.BlockSpec((1, 2 * c, d), lambda i: (i, 0, 0)),
        out_shape=jax.ShapeDtypeStruct((b, 2 * c, d), jnp.float32),
        compiler_params=pltpu.CompilerParams(
            dimension_semantics=("parallel",),
        ),
    )(r, cues, patches)
    return out


# speedup vs baseline: 1.3390x; 1.0113x over previous
import jax
import jax.numpy as jnp
from jax.experimental import pallas as pl
from jax.experimental.pallas import tpu as pltpu

_B, _R, _D = 64, 4, 768
_H = _W = 37
_HW = _H * _W
_CH = 384
_NCH = 4


def _encoder_body(r_ref, cues_ref, p0, p1, p2, p3, out_ref):
    r = r_ref[0]
    cues = cues_ref[0]
    c = cues.shape[0]
    chunks = [p0[0], p1[0], p2[0], p3[0]]

    tail_rows = _HW - (_NCH - 1) * _CH
    row_last = jax.lax.broadcasted_iota(jnp.int32, (_CH, 1), 0)
    chunks[-1] = jnp.where(row_last < tail_rows, chunks[-1], 0.0)

    sims = [
        jax.lax.dot_general(cues, ch, (((1,), (1,)), ((), ())),
                            preferred_element_type=jnp.float32)
        for ch in chunks
    ]
    sim = jnp.concatenate(sims, axis=1)
    pos = jax.lax.broadcasted_iota(jnp.int32, (c, _NCH * _CH), 1)
    sim = jnp.where(pos < _HW, sim, -jnp.inf)
    idx = jnp.argmax(sim, axis=1, keepdims=True)
    hh = idx // _W
    ww = idx % _W

    num = jnp.zeros((c, _D), jnp.float32)
    for j, ch in enumerate(chunks):
        posj = jax.lax.broadcasted_iota(jnp.int32, (c, _CH), 1) + j * _CH
        rowp = posj // _W
        colp = posj % _W
        inside = ((jnp.abs(rowp - hh) <= r) & (jnp.abs(colp - ww) <= r)
                  & (posj < _HW))
        maskf = jnp.where(inside, 1.0, 0.0)
        num = num + jax.lax.dot_general(
            maskf, ch, (((1,), (0,)), ((), ())),
            preferred_element_type=jnp.float32)

    nrows = jnp.minimum(hh + r, _H - 1) - jnp.maximum(hh - r, 0) + 1
    ncols = jnp.minimum(ww + r, _W - 1) - jnp.maximum(ww - r, 0) + 1
    cnt = (nrows * ncols).astype(jnp.float32)
    rois = num / cnt

    toks = jnp.concatenate([cues, rois], axis=0)
    nrm = jnp.sqrt(jnp.sum(toks * toks, axis=1, keepdims=True))
    out_ref[0] = toks / jnp.maximum(nrm, 1e-12)


def kernel(cls_tok, regs, patches2d, roi_side):
    b, h, w, d = patches2d.shape
    c = 1 + regs.shape[1]
    cues = jnp.concatenate([cls_tok[:, None, :], regs], axis=1)
    patches = patches2d.reshape(b, h * w, d)
    r = jnp.asarray(roi_side // 2, jnp.int32).reshape(1)

    patch_specs = [
        pl.BlockSpec((1, _CH, d), lambda i, j=j: (i, j, 0))
        for j in range(_NCH)
    ]
    out = pl.pallas_call(
        _encoder_body,
        grid=(b,),
        in_specs=[
            pl.BlockSpec(memory_space=pltpu.SMEM),
            pl.BlockSpec((1, c, d), lambda i: (i, 0, 0)),
            *patch_specs,
        ],
        out_specs=pl.BlockSpec((1, 2 * c, d), lambda i: (i, 0, 0)),
        out_shape=jax.ShapeDtypeStruct((b, 2 * c, d), jnp.float32),
        compiler_params=pltpu.CompilerParams(
            dimension_semantics=("parallel",),
        ),
    )(r, cues, patches, patches, patches, patches)
    return out


# speedup vs baseline: 1.3396x; 1.0004x over previous
import jax
import jax.numpy as jnp
from jax.experimental import pallas as pl
from jax.experimental.pallas import tpu as pltpu

_B, _R, _D = 64, 4, 768
_H = _W = 37
_HW = _H * _W
_CH = 384
_NCH = 4


def _encoder_body(r_ref, cues_ref, p0, p1, p2, p3, out_ref):
    r = r_ref[0]
    cues = cues_ref[0]
    c = cues.shape[0]
    chunks = [p0[0], p1[0], p2[0], p3[0]]

    tail_rows = _HW - (_NCH - 1) * _CH
    row_last = jax.lax.broadcasted_iota(jnp.int32, (_CH, 1), 0)
    chunks[-1] = jnp.where(row_last < tail_rows, chunks[-1], 0.0)

    sims = [
        jax.lax.dot_general(cues, ch, (((1,), (1,)), ((), ())),
                            preferred_element_type=jnp.float32)
        for ch in chunks
    ]
    sim = jnp.concatenate(sims, axis=1)
    pos = jax.lax.broadcasted_iota(jnp.int32, (c, _NCH * _CH), 1)
    sim = jnp.where(pos < _HW, sim, -jnp.inf)
    idx = jnp.argmax(sim, axis=1, keepdims=True)
    hh = idx // _W
    ww = idx % _W

    num = jnp.zeros((c, _D), jnp.float32)
    for j, ch in enumerate(chunks):
        posj = jax.lax.broadcasted_iota(jnp.int32, (c, _CH), 1) + j * _CH
        rowp = posj // _W
        colp = posj % _W
        inside = ((jnp.abs(rowp - hh) <= r) & (jnp.abs(colp - ww) <= r)
                  & (posj < _HW))
        maskf = jnp.where(inside, 1.0, 0.0)
        num = num + jax.lax.dot_general(
            maskf, ch, (((1,), (0,)), ((), ())),
            preferred_element_type=jnp.float32)

    nrows = jnp.minimum(hh + r, _H - 1) - jnp.maximum(hh - r, 0) + 1
    ncols = jnp.minimum(ww + r, _W - 1) - jnp.maximum(ww - r, 0) + 1
    cnt = (nrows * ncols).astype(jnp.float32)
    rois = num / cnt

    toks = jnp.concatenate([cues, rois], axis=0)
    nrm = jnp.sqrt(jnp.sum(toks * toks, axis=1, keepdims=True))
    out_ref[0] = toks / jnp.maximum(nrm, 1e-12)


def kernel(cls_tok, regs, patches2d, roi_side):
    b, h, w, d = patches2d.shape
    c = 1 + regs.shape[1]
    cues = jnp.concatenate([cls_tok[:, None, :], regs], axis=1)
    patches = patches2d.reshape(b, h * w, d)
    r = jnp.asarray(roi_side // 2, jnp.int32).reshape(1)

    patch_specs = [
        pl.BlockSpec((1, _CH, d), lambda i, j=j: (i, j, 0))
        for j in range(_NCH)
    ]
    out = pl.pallas_call(
        _encoder_body,
        grid=(b,),
        in_specs=[
            pl.BlockSpec(memory_space=pltpu.SMEM),
            pl.BlockSpec((1, c, d), lambda i: (i, 0, 0)),
            *patch_specs,
        ],
        out_specs=pl.BlockSpec((1, 2 * c, d), lambda i: (i, 0, 0)),
        out_shape=jax.ShapeDtypeStruct((b, 2 * c, d), jnp.float32),
        compiler_params=pltpu.CompilerParams(
            dimension_semantics=("arbitrary",),
        ),
    )(r, cues, patches, patches, patches, patches)
    return out
